# baseline (speedup 1.0000x reference)
import functools

import jax
import jax.numpy as jnp
from jax import lax
from jax.experimental import pallas as pl
from jax.experimental.pallas import tpu as pltpu
from jax.experimental.pallas import tpu_sc as plsc


def _dist_tile_kernel(x1_ref, x2_ref, out_ref):
    a = x1_ref[...]
    b = x2_ref[...]
    g = jax.lax.dot_general(
        a, b, (((1,), (1,)), ((), ())), preferred_element_type=jnp.float32
    )
    na = jnp.sum(a * a, axis=1, keepdims=True)
    nb = jnp.sum(b * b, axis=1, keepdims=True).T
    d2 = na + nb - 2.0 * g
    out_ref[...] = jnp.sqrt(jnp.maximum(d2, 0.0) + 1e-12)


@functools.partial(jax.jit, static_argnames=("bm",))
def _pairwise_dist_tc(x1, x2, bm=512):
    n1, d = x1.shape
    n2 = x2.shape[0]
    grid = (n1 // bm,)
    return pl.pallas_call(
        _dist_tile_kernel,
        grid=grid,
        in_specs=[
            pl.BlockSpec((bm, d), lambda i: (i, 0)),
            pl.BlockSpec((n2, d), lambda i: (0, 0)),
        ],
        out_specs=pl.BlockSpec((bm, n2), lambda i: (i, 0)),
        out_shape=jax.ShapeDtypeStruct((n1, n2), jnp.float32),
    )(x1, x2)


_NC, _NS, _L = 2, 16, 16
_NW = _NC * _NS


def _sqrt_newton(x):
    i = lax.bitcast_convert_type(x, jnp.int32)
    y = lax.bitcast_convert_type(jnp.int32(0x5F3759DF) - (i >> 1), jnp.float32)
    for _ in range(3):
        y = y * (1.5 - 0.5 * x * y * y)
    return x * y


def _make_sc_dist(n1, n2, d):
    assert n1 % _NW == 0 and n2 % _L == 0
    rows = n1 // _NW
    mesh = plsc.VectorSubcoreMesh(core_axis_name="c", subcore_axis_name="s")

    @functools.partial(
        pl.kernel,
        out_type=jax.ShapeDtypeStruct((n1, n2), jnp.float32),
        mesh=mesh,
        scratch_types=[
            pltpu.VMEM((rows * d,), jnp.float32),
            pltpu.VMEM((d, n2), jnp.float32),
            pltpu.VMEM((rows, n2), jnp.float32),
        ],
    )
    def sc_dist(x1_hbm, x2t_hbm, out_hbm, x1_v, x2t_v, out_v):
        wid = lax.axis_index("s") * _NC + lax.axis_index("c")
        base = wid * rows
        pltpu.sync_copy(x2t_hbm, x2t_v)
        pltpu.sync_copy(x1_hbm.at[pl.ds(base * d, rows * d)], x1_v)

        def i_body(i, _):
            def jc_body(jc, _):
                js = jc * _L
                acc = jnp.zeros((_L,), jnp.float32)
                for kg in range(d // _L):
                    xv = x1_v[pl.ds(i * d + kg * _L, _L)]
                    for kk in range(_L):
                        a = lax.gather(
                            xv,
                            jnp.full((_L, 1), kk, jnp.int32),
                            lax.GatherDimensionNumbers(
                                offset_dims=(),
                                collapsed_slice_dims=(0,),
                                start_index_map=(0,),
                            ),
                            slice_sizes=(1,),
                            mode=lax.GatherScatterMode.PROMISE_IN_BOUNDS,
                        )
                        b = x2t_v[kg * _L + kk, pl.ds(js, _L)]
                        diff = b - a
                        acc = acc + diff * diff
                out_v[i, pl.ds(js, _L)] = _sqrt_newton(acc + 1e-12)
                return 0

            return lax.fori_loop(0, n2 // _L, jc_body, 0)

        lax.fori_loop(0, rows, i_body, 0)
        pltpu.sync_copy(out_v, out_hbm.at[pl.ds(base, rows)])

    return sc_dist


@jax.jit
def _pairwise_dist_sc(x1, x2):
    n1, d = x1.shape
    n2 = x2.shape[0]
    x2t = x2.T
    return _make_sc_dist(n1, n2, d)(x1.reshape(-1), x2t)


def kernel(x1, x2):
    return _pairwise_dist_sc(x1, x2)


# speedup vs baseline: 5.4064x; 5.4064x over previous
import functools

import jax
import jax.numpy as jnp
from jax import lax
from jax.experimental import pallas as pl
from jax.experimental.pallas import tpu as pltpu
from jax.experimental.pallas import tpu_sc as plsc


def _dist_tile_kernel(x1_ref, x2_ref, out_ref):
    a = x1_ref[...]
    b = x2_ref[...]
    g = jax.lax.dot_general(
        a, b, (((1,), (1,)), ((), ())), preferred_element_type=jnp.float32
    )
    na = jnp.sum(a * a, axis=1, keepdims=True)
    nb = jnp.sum(b * b, axis=1, keepdims=True).T
    d2 = na + nb - 2.0 * g
    out_ref[...] = jnp.sqrt(jnp.maximum(d2, 0.0) + 1e-12)


@functools.partial(jax.jit, static_argnames=("bm",))
def _pairwise_dist_tc(x1, x2, bm=512):
    n1, d = x1.shape
    n2 = x2.shape[0]
    grid = (n1 // bm,)
    return pl.pallas_call(
        _dist_tile_kernel,
        grid=grid,
        in_specs=[
            pl.BlockSpec((bm, d), lambda i: (i, 0)),
            pl.BlockSpec((n2, d), lambda i: (0, 0)),
        ],
        out_specs=pl.BlockSpec((bm, n2), lambda i: (i, 0)),
        out_shape=jax.ShapeDtypeStruct((n1, n2), jnp.float32),
    )(x1, x2)


_NC, _NS, _L = 2, 16, 16
_NW = _NC * _NS


def _sqrt_newton(x):
    i = lax.bitcast_convert_type(x, jnp.int32)
    y = lax.bitcast_convert_type(jnp.int32(0x5F3759DF) - (i >> 1), jnp.float32)
    for _ in range(3):
        y = y * (1.5 - 0.5 * x * y * y)
    return x * y


def _make_sc_dist(n1, n2, d):
    assert n1 % _NW == 0 and n2 % _L == 0
    rows = n1 // _NW
    mesh = plsc.VectorSubcoreMesh(core_axis_name="c", subcore_axis_name="s")

    @functools.partial(
        pl.kernel,
        out_type=jax.ShapeDtypeStruct((n1, n2), jnp.float32),
        mesh=mesh,
        scratch_types=[
            pltpu.VMEM((rows * d,), jnp.float32),
            pltpu.VMEM((d, n2), jnp.float32),
            pltpu.VMEM((rows, n2), jnp.float32),
        ],
    )
    def sc_dist(x1_hbm, x2t_hbm, out_hbm, x1_v, x2t_v, out_v):
        wid = lax.axis_index("s") * _NC + lax.axis_index("c")
        base = wid * rows
        pltpu.sync_copy(x2t_hbm, x2t_v)
        pltpu.sync_copy(x1_hbm.at[pl.ds(base * d, rows * d)], x1_v)

        def i_body(i, _):
            def jc_body(jc, _):
                js = jc * _L
                acc = jnp.zeros((_L,), jnp.float32)
                for kg in range(d // _L):
                    xv = x1_v[pl.ds(i * d + kg * _L, _L)]
                    for kk in range(_L):
                        a = lax.gather(
                            xv,
                            jnp.full((_L, 1), kk, jnp.int32),
                            lax.GatherDimensionNumbers(
                                offset_dims=(),
                                collapsed_slice_dims=(0,),
                                start_index_map=(0,),
                            ),
                            slice_sizes=(1,),
                            mode=lax.GatherScatterMode.PROMISE_IN_BOUNDS,
                        )
                        b = x2t_v[kg * _L + kk, pl.ds(js, _L)]
                        diff = b - a
                        acc = acc + diff * diff
                out_v[i, pl.ds(js, _L)] = _sqrt_newton(acc + 1e-12)
                return 0

            return lax.fori_loop(0, n2 // _L, jc_body, 0)

        lax.fori_loop(0, rows, i_body, 0)
        pltpu.sync_copy(out_v, out_hbm.at[pl.ds(base, rows)])

    return sc_dist


@jax.jit
def _pairwise_dist_sc(x1, x2):
    n1, d = x1.shape
    n2 = x2.shape[0]
    x2t = x2.T
    return _make_sc_dist(n1, n2, d)(x1.reshape(-1), x2t)


@functools.partial(jax.jit, static_argnames=("sc_rows",))
def _pairwise_dist_hybrid(x1, x2, sc_rows=32):
    n1 = x1.shape[0]
    n_tc = n1 - sc_rows
    d_tc = _pairwise_dist_tc(x1[:n_tc], x2, bm=n_tc // 2)
    d_sc = _pairwise_dist_sc(x1[n_tc:], x2)
    return jnp.concatenate([d_tc, d_sc], axis=0)


def kernel(x1, x2):
    return _pairwise_dist_hybrid(x1, x2)


# speedup vs baseline: 23.0160x; 4.2572x over previous
import functools

import jax
import jax.numpy as jnp
from jax import lax
from jax.experimental import pallas as pl
from jax.experimental.pallas import tpu as pltpu
from jax.experimental.pallas import tpu_sc as plsc


def _dist_tile_kernel(x1_ref, x2_ref, out_ref):
    a = x1_ref[...]
    b = x2_ref[...]
    g = jax.lax.dot_general(
        a, b, (((1,), (1,)), ((), ())), preferred_element_type=jnp.float32
    )
    na = jnp.sum(a * a, axis=1, keepdims=True)
    nb = jnp.sum(b * b, axis=1, keepdims=True).T
    d2 = na + nb - 2.0 * g
    out_ref[...] = jnp.sqrt(jnp.maximum(d2, 0.0) + 1e-12)


@functools.partial(jax.jit, static_argnames=("bm",))
def _pairwise_dist_tc(x1, x2, bm=512):
    n1, d = x1.shape
    n2 = x2.shape[0]
    grid = (n1 // bm,)
    return pl.pallas_call(
        _dist_tile_kernel,
        grid=grid,
        in_specs=[
            pl.BlockSpec((bm, d), lambda i: (i, 0)),
            pl.BlockSpec((n2, d), lambda i: (0, 0)),
        ],
        out_specs=pl.BlockSpec((bm, n2), lambda i: (i, 0)),
        out_shape=jax.ShapeDtypeStruct((n1, n2), jnp.float32),
    )(x1, x2)


_NC, _NS, _L = 2, 16, 16
_NW = _NC * _NS


def _sqrt_newton(x):
    i = lax.bitcast_convert_type(x, jnp.int32)
    y = lax.bitcast_convert_type(jnp.int32(0x5F3759DF) - (i >> 1), jnp.float32)
    for _ in range(3):
        y = y * (1.5 - 0.5 * x * y * y)
    return x * y


def _make_sc_dist(n1, n2, d):
    assert n1 % _NW == 0 and n2 % _L == 0
    rows = n1 // _NW
    mesh = plsc.VectorSubcoreMesh(core_axis_name="c", subcore_axis_name="s")

    @functools.partial(
        pl.kernel,
        out_type=jax.ShapeDtypeStruct((n1, n2), jnp.float32),
        mesh=mesh,
        scratch_types=[
            pltpu.VMEM((rows * d,), jnp.float32),
            pltpu.VMEM((d, n2), jnp.float32),
            pltpu.VMEM((rows, n2), jnp.float32),
        ],
    )
    def sc_dist(x1_hbm, x2t_hbm, out_hbm, x1_v, x2t_v, out_v):
        wid = lax.axis_index("s") * _NC + lax.axis_index("c")
        base = wid * rows
        pltpu.sync_copy(x2t_hbm, x2t_v)
        pltpu.sync_copy(x1_hbm.at[pl.ds(base * d, rows * d)], x1_v)

        def i_body(i, _):
            def jc_body(jc, _):
                js = jc * _L
                acc = jnp.zeros((_L,), jnp.float32)
                for kg in range(d // _L):
                    xv = x1_v[pl.ds(i * d + kg * _L, _L)]
                    for kk in range(_L):
                        a = lax.gather(
                            xv,
                            jnp.full((_L, 1), kk, jnp.int32),
                            lax.GatherDimensionNumbers(
                                offset_dims=(),
                                collapsed_slice_dims=(0,),
                                start_index_map=(0,),
                            ),
                            slice_sizes=(1,),
                            mode=lax.GatherScatterMode.PROMISE_IN_BOUNDS,
                        )
                        b = x2t_v[kg * _L + kk, pl.ds(js, _L)]
                        diff = b - a
                        acc = acc + diff * diff
                out_v[i, pl.ds(js, _L)] = _sqrt_newton(acc + 1e-12)
                return 0

            return lax.fori_loop(0, n2 // _L, jc_body, 0)

        lax.fori_loop(0, rows, i_body, 0)
        pltpu.sync_copy(out_v, out_hbm.at[pl.ds(base, rows)])

    return sc_dist


@jax.jit
def _pairwise_dist_sc(x1, x2):
    n1, d = x1.shape
    n2 = x2.shape[0]
    x2t = x2.T
    return _make_sc_dist(n1, n2, d)(x1.reshape(-1), x2t)


@functools.partial(jax.jit, static_argnames=("sc_rows",))
def _pairwise_dist_hybrid(x1, x2, sc_rows=32):
    n1 = x1.shape[0]
    n_tc = n1 - sc_rows
    d_tc = _pairwise_dist_tc(x1[:n_tc], x2, bm=n_tc // 2)
    d_sc = _pairwise_dist_sc(x1[n_tc:], x2)
    return jnp.concatenate([d_tc, d_sc], axis=0)


def kernel(x1, x2):
    return _pairwise_dist_tc(x1, x2)
